# baseline (speedup 1.0000x reference)
import functools

import jax
import jax.numpy as jnp
import numpy as np
from jax import lax
from jax.experimental import pallas as pl
from jax.experimental.pallas import tpu as pltpu
from jax.experimental.pallas import tpu_sc as plsc

N_NODES = 10000
D = 128
N_EDGES = 320000
N_PAD = 10112

NC = 2
NS = 16
L = 16

CHUNK = 112
NW = NC * NS
NBUF = 2
CHUNKS_PER_TILE = 90
E_PAD = NW * CHUNK * CHUNKS_PER_TILE
N_CHUNKS = E_PAD // CHUNK
ROWS_PER_TILE = N_PAD // NS

_MASK_HI = np.int32(-65536)

_TP = np.zeros((D,), np.int32)
for _b in range(D // 32):
    for _i in range(16):
        _TP[32 * _b + 2 * _i] = 32 * _b + _i
        _TP[32 * _b + 2 * _i + 1] = 32 * _b + 16 + _i

_mesh = plsc.VectorSubcoreMesh(core_axis_name="c", subcore_axis_name="s")

_GDN = lax.GatherDimensionNumbers(
    offset_dims=(), collapsed_slice_dims=(0,), start_index_map=(0,))


def _bcast_lane(vec, j):
    idx = jnp.full((L, 1), j, jnp.int32)
    return lax.gather(vec, idx, _GDN, (1,),
                      mode=lax.GatherScatterMode.PROMISE_IN_BOUNDS)


def _pack_table(h):
    ht = h[:, _TP].astype(jnp.bfloat16)
    return lax.bitcast_convert_type(ht.reshape(-1, D // 2, 2), jnp.int32)


@functools.partial(
    pl.kernel,
    out_type=jax.ShapeDtypeStruct((NC, N_PAD, D), jnp.float32),
    mesh=_mesh,
    scratch_types=[
        pltpu.VMEM((NBUF, 3, CHUNK), jnp.int32),
        pltpu.VMEM((NBUF, CHUNK), jnp.int32),
        pltpu.VMEM((NBUF, CHUNK, D // 2), jnp.int32),
        pltpu.VMEM((NBUF, CHUNK, D), jnp.float32),
        pltpu.VMEM_SHARED((N_PAD, D), jnp.float32),
    ] + [pltpu.SemaphoreType.DMA] * (3 * NBUF),
    compiler_params=pltpu.CompilerParams(use_tc_tiling_on_sc=False),
)
def _layer(table_hbm, edges_hbm, out_hbm, edge_v, dst_v, bf_v, rows_v, acc,
           *sems):
    esem = sems[:NBUF]
    gsem = sems[NBUF:2 * NBUF]
    ssem = sems[2 * NBUF:]
    c = lax.axis_index("c")
    s = lax.axis_index("s")
    wid = c * NS + s

    zeros16 = jnp.zeros((L,), jnp.float32)

    def _zrow(i, _):
        for cb in range(D // L):
            rows_v[0, i, pl.ds(cb * L, L)] = zeros16
        return 0
    lax.fori_loop(0, CHUNK, _zrow, 0)

    row_base = s * ROWS_PER_TILE
    for k in range(5):
        pltpu.sync_copy(rows_v.at[0], acc.at[pl.ds(row_base + k * CHUNK, CHUNK)])
    rem = ROWS_PER_TILE - 5 * CHUNK
    pltpu.sync_copy(rows_v.at[0, pl.ds(0, rem)],
                    acc.at[pl.ds(row_base + 5 * CHUNK, rem)])
    plsc.subcore_barrier()

    chunk_base = wid * CHUNKS_PER_TILE

    def _edge_start(slot, g):
        pltpu.async_copy(edges_hbm.at[chunk_base + g], edge_v.at[slot],
                         esem[slot])

    def _edge_wait(slot):
        pltpu.make_async_copy(edges_hbm.at[chunk_base], edge_v.at[slot],
                              esem[slot]).wait()

    def _gather_start(slot):
        pltpu.async_copy(table_hbm.at[edge_v.at[slot, 0]], bf_v.at[slot],
                         gsem[slot])

    def _gather_wait(slot):
        pltpu.make_async_copy(table_hbm.at[edge_v.at[slot, 0]],
                              bf_v.at[slot], gsem[slot]).wait()

    def _scatter_start(slot):
        pltpu.async_copy(rows_v.at[slot], acc.at[dst_v.at[slot]],
                         ssem[slot], add=True)

    def _scatter_wait(slot):
        pltpu.make_async_copy(rows_v.at[slot], acc.at[dst_v.at[slot]],
                              ssem[slot]).wait()

    _edge_start(0, 0)
    _edge_start(1, 1)
    _edge_wait(0)
    _gather_start(0)

    def _visit(g, slot):
        other = 1 - slot

        @pl.when(g >= 2)
        def _():
            _scatter_wait(slot)

        @pl.when(g + 1 < CHUNKS_PER_TILE)
        def _():
            _edge_wait(other)
            _gather_start(other)

        _gather_wait(slot)

        for q in range(CHUNK // L):
            dst_v[slot, pl.ds(q * L, L)] = edge_v[slot, 1, pl.ds(q * L, L)]

        def _scale(gg, _):
            ewv = lax.bitcast_convert_type(
                edge_v[slot, 2, pl.ds(gg * L, L)], jnp.float32)
            for j in range(L):
                ewb = _bcast_lane(ewv, j)
                r = gg * L + j
                for blk in range(D // 32):
                    w = bf_v[slot, r, pl.ds(blk * L, L)]
                    lo = lax.bitcast_convert_type(w << 16, jnp.float32)
                    hi = lax.bitcast_convert_type(w & _MASK_HI, jnp.float32)
                    rows_v[slot, r, pl.ds(blk * 32, L)] = lo * ewb
                    rows_v[slot, r, pl.ds(blk * 32 + L, L)] = hi * ewb
            return 0
        lax.fori_loop(0, CHUNK // L, _scale, 0)

        _scatter_start(slot)

        @pl.when(g + 2 < CHUNKS_PER_TILE)
        def _():
            _edge_start(slot, g + 2)

    def _round(og, _):
        for b in range(NBUF):
            _visit(og * NBUF + b, b)
        return 0
    lax.fori_loop(0, CHUNKS_PER_TILE // NBUF, _round, 0)

    _scatter_wait((CHUNKS_PER_TILE - 2) % NBUF)
    _scatter_wait((CHUNKS_PER_TILE - 1) % NBUF)
    plsc.subcore_barrier()

    pltpu.sync_copy(acc.at[pl.ds(row_base, ROWS_PER_TILE)],
                    out_hbm.at[c, pl.ds(row_base, ROWS_PER_TILE)])


def _sum2_body(p_ref, o_ref):
    o_ref[...] = p_ref[0] + p_ref[1]


def _final_body(x_ref, h1_ref, q_ref, o_ref):
    o_ref[...] = (x_ref[...] + h1_ref[...] + q_ref[0] + q_ref[1]) * (1.0 / 3.0)


_RB = 2000
_RB2 = 1264

_sum2 = pl.pallas_call(
    _sum2_body,
    grid=(N_PAD // _RB2,),
    in_specs=[pl.BlockSpec((NC, _RB2, D), lambda i: (0, i, 0))],
    out_specs=pl.BlockSpec((_RB2, D), lambda i: (i, 0)),
    out_shape=jax.ShapeDtypeStruct((N_PAD, D), jnp.float32),
)

_final = pl.pallas_call(
    _final_body,
    grid=(N_NODES // _RB,),
    in_specs=[
        pl.BlockSpec((_RB, D), lambda i: (i, 0)),
        pl.BlockSpec((_RB, D), lambda i: (i, 0)),
        pl.BlockSpec((NC, _RB, D), lambda i: (0, i, 0)),
    ],
    out_specs=pl.BlockSpec((_RB, D), lambda i: (i, 0)),
    out_shape=jax.ShapeDtypeStruct((N_NODES, D), jnp.float32),
)


@jax.jit
def kernel(x, edge_index, edge_weight):
    src = edge_index[0].astype(jnp.int32)
    dst = edge_index[1].astype(jnp.int32)
    ew = edge_weight.astype(jnp.float32)

    pad = E_PAD - N_EDGES
    spread = jnp.arange(pad, dtype=jnp.int32) % N_NODES
    src = jnp.concatenate([src, spread])
    dst = jnp.concatenate([dst, spread])
    ew = jnp.concatenate([ew, jnp.zeros((pad,), jnp.float32)])
    ew_bits = lax.bitcast_convert_type(ew, jnp.int32)
    edges = jnp.stack([src.reshape(N_CHUNKS, CHUNK),
                       dst.reshape(N_CHUNKS, CHUNK),
                       ew_bits.reshape(N_CHUNKS, CHUNK)], axis=1)

    p = _layer(_pack_table(x), edges)
    h1 = _sum2(p)
    q = _layer(_pack_table(h1), edges)
    return _final(x, h1, q)


# speedup vs baseline: 2.3320x; 2.3320x over previous
import functools

import jax
import jax.numpy as jnp
from jax import lax
from jax.experimental import pallas as pl
from jax.experimental.pallas import tpu as pltpu
from jax.experimental.pallas import tpu_sc as plsc

N_NODES = 10000
D = 128
N_EDGES = 320000
N_PAD = 10240

NC = 2
NS = 16
L = 16

CHUNK = 128
NW = NC * NS
NBUF = 2
CHUNKS_PER_TILE = 80
E_PAD = NW * CHUNK * CHUNKS_PER_TILE
N_CHUNKS = E_PAD // CHUNK
ROWS_PER_TILE = N_PAD // NS

_mesh = plsc.VectorSubcoreMesh(core_axis_name="c", subcore_axis_name="s")

_GDN = lax.GatherDimensionNumbers(
    offset_dims=(), collapsed_slice_dims=(0,), start_index_map=(0,))


def _bcast_lane(vec, j):
    idx = jnp.full((L, 1), j, jnp.int32)
    return lax.gather(vec, idx, _GDN, (1,),
                      mode=lax.GatherScatterMode.PROMISE_IN_BOUNDS)


@functools.partial(
    pl.kernel,
    out_type=jax.ShapeDtypeStruct((NC, N_PAD, D), jnp.float32),
    mesh=_mesh,
    scratch_types=[
        pltpu.VMEM((NBUF, 3, CHUNK), jnp.int32),
        pltpu.VMEM((NBUF, CHUNK), jnp.int32),
        pltpu.VMEM((NBUF, CHUNK, D), jnp.float32),
        pltpu.VMEM_SHARED((N_PAD, D), jnp.float32),
    ] + [pltpu.SemaphoreType.DMA] * (3 * NBUF),
)
def _layer(table_hbm, edges_hbm, out_hbm, edge_v, dst_v, rows_v, acc, *sems):
    esem = sems[:NBUF]
    gsem = sems[NBUF:2 * NBUF]
    ssem = sems[2 * NBUF:]
    c = lax.axis_index("c")
    s = lax.axis_index("s")
    wid = c * NS + s

    zeros16 = jnp.zeros((L,), jnp.float32)

    def _zrow(i, _):
        for cb in range(D // L):
            rows_v[0, i, pl.ds(cb * L, L)] = zeros16
        return 0
    lax.fori_loop(0, CHUNK, _zrow, 0)

    row_base = s * ROWS_PER_TILE
    for k in range(ROWS_PER_TILE // CHUNK):
        pltpu.sync_copy(rows_v.at[0], acc.at[pl.ds(row_base + k * CHUNK, CHUNK)])
    plsc.subcore_barrier()

    chunk_base = wid * CHUNKS_PER_TILE

    def _edge_start(slot, g):
        pltpu.async_copy(edges_hbm.at[chunk_base + g], edge_v.at[slot],
                         esem[slot])

    def _edge_wait(slot):
        pltpu.make_async_copy(edges_hbm.at[chunk_base], edge_v.at[slot],
                              esem[slot]).wait()

    def _gather_start(slot):
        pltpu.async_copy(table_hbm.at[edge_v.at[slot, 0]], rows_v.at[slot],
                         gsem[slot])

    def _gather_wait(slot):
        pltpu.make_async_copy(table_hbm.at[edge_v.at[slot, 0]],
                              rows_v.at[slot], gsem[slot]).wait()

    def _scatter_start(slot):
        pltpu.async_copy(rows_v.at[slot], acc.at[dst_v.at[slot]],
                         ssem[slot], add=True)

    def _scatter_wait(slot):
        pltpu.make_async_copy(rows_v.at[slot], acc.at[dst_v.at[slot]],
                              ssem[slot]).wait()

    _edge_start(0, 0)
    _edge_start(1, 1)
    _edge_wait(0)
    _gather_start(0)

    def _visit(g, slot):
        other = 1 - slot

        @pl.when(g >= 2)
        def _():
            _scatter_wait(slot)

        @pl.when(g + 1 < CHUNKS_PER_TILE)
        def _():
            _edge_wait(other)
            _gather_start(other)

        _gather_wait(slot)

        for q in range(CHUNK // L):
            dst_v[slot, pl.ds(q * L, L)] = edge_v[slot, 1, pl.ds(q * L, L)]

        def _scale(gg, _):
            ewv = lax.bitcast_convert_type(
                edge_v[slot, 2, pl.ds(gg * L, L)], jnp.float32)
            for j in range(L):
                ewb = _bcast_lane(ewv, j)
                r = gg * L + j
                for cb in range(D // L):
                    sl = pl.ds(cb * L, L)
                    rows_v[slot, r, sl] = rows_v[slot, r, sl] * ewb
            return 0
        lax.fori_loop(0, CHUNK // L, _scale, 0)

        _scatter_start(slot)

        @pl.when(g + 2 < CHUNKS_PER_TILE)
        def _():
            _edge_start(slot, g + 2)

    def _round(og, _):
        for b in range(NBUF):
            _visit(og * NBUF + b, b)
        return 0
    lax.fori_loop(0, CHUNKS_PER_TILE // NBUF, _round, 0)

    _scatter_wait((CHUNKS_PER_TILE - 2) % NBUF)
    _scatter_wait((CHUNKS_PER_TILE - 1) % NBUF)
    plsc.subcore_barrier()

    pltpu.sync_copy(acc.at[pl.ds(row_base, ROWS_PER_TILE)],
                    out_hbm.at[c, pl.ds(row_base, ROWS_PER_TILE)])


def _sum2_body(p_ref, o_ref):
    o_ref[...] = p_ref[0] + p_ref[1]


def _final_body(x_ref, h1_ref, q_ref, o_ref):
    o_ref[...] = (x_ref[...] + h1_ref[...] + q_ref[0] + q_ref[1]) * (1.0 / 3.0)


_RB = 2000
_RB2 = 1280

_sum2 = pl.pallas_call(
    _sum2_body,
    grid=(N_PAD // _RB2,),
    in_specs=[pl.BlockSpec((NC, _RB2, D), lambda i: (0, i, 0))],
    out_specs=pl.BlockSpec((_RB2, D), lambda i: (i, 0)),
    out_shape=jax.ShapeDtypeStruct((N_PAD, D), jnp.float32),
)

_final = pl.pallas_call(
    _final_body,
    grid=(N_NODES // _RB,),
    in_specs=[
        pl.BlockSpec((_RB, D), lambda i: (i, 0)),
        pl.BlockSpec((_RB, D), lambda i: (i, 0)),
        pl.BlockSpec((NC, _RB, D), lambda i: (0, i, 0)),
    ],
    out_specs=pl.BlockSpec((_RB, D), lambda i: (i, 0)),
    out_shape=jax.ShapeDtypeStruct((N_NODES, D), jnp.float32),
)


@jax.jit
def kernel(x, edge_index, edge_weight):
    src = edge_index[0].astype(jnp.int32)
    dst = edge_index[1].astype(jnp.int32)
    ew = edge_weight.astype(jnp.float32)

    pad = E_PAD - N_EDGES
    spread = jnp.arange(pad, dtype=jnp.int32) % N_NODES
    src = jnp.concatenate([src, spread])
    dst = jnp.concatenate([dst, spread])
    ew = jnp.concatenate([ew, jnp.zeros((pad,), jnp.float32)])
    ew_bits = lax.bitcast_convert_type(ew, jnp.int32)
    edges = jnp.stack([src.reshape(N_CHUNKS, CHUNK),
                       dst.reshape(N_CHUNKS, CHUNK),
                       ew_bits.reshape(N_CHUNKS, CHUNK)], axis=1)

    p = _layer(x, edges)
    h1 = _sum2(p)
    q = _layer(h1, edges)
    return _final(x, h1, q)
